# baseline (speedup 1.0000x reference)
import functools

import jax
import jax.numpy as jnp
from jax import lax
from jax.experimental import pallas as pl
from jax.experimental.pallas import tpu as pltpu
from jax.experimental.pallas import tpu_sc as plsc

T = 2048
D = 1024
H = 16
HD = 64
DFF = 2048
E = 8
EPS = 1e-5
SCALE = HD ** -0.5

TILE = 128
NTILES = 40
PADDED = NTILES * TILE
NPAIR = 2 * T

BQ = 1024
BR = 512

_PREC = None
_XPREC = jax.lax.Precision.HIGHEST


def _qkv_body(x_ref, wq_ref, wk_ref, wv_ref, b_ref, o_ref):
    j = pl.program_id(1)
    for idx in range(3):
        w_ref = (wq_ref, wk_ref, wv_ref)[idx]

        @pl.when(j == idx)
        def _(w_ref=w_ref):
            o_ref[...] = (
                lax.dot_general(x_ref[...], w_ref[...],
                                (((1,), (1,)), ((), ())),
                                preferred_element_type=jnp.float32,
                                precision=_PREC)
                + b_ref[0]
            ).astype(jnp.bfloat16)


def _qkv_call(x, wq, wk, wv, b3):
    return pl.pallas_call(
        _qkv_body,
        grid=(T // BR, 3),
        in_specs=[
            pl.BlockSpec((BR, D), lambda i, j: (i, 0)),
            pl.BlockSpec((D, D), lambda i, j: (0, 0)),
            pl.BlockSpec((D, D), lambda i, j: (0, 0)),
            pl.BlockSpec((D, D), lambda i, j: (0, 0)),
            pl.BlockSpec((1, 1, D), lambda i, j: (j, 0, 0)),
        ],
        out_specs=pl.BlockSpec((BR, D), lambda i, j: (i, j)),
        out_shape=jax.ShapeDtypeStruct((T, 3 * D), jnp.bfloat16),
    )(x, wq, wk, wv, b3)


def _attn_body(ap_ref, an_ref, fq_ref, fk_ref, q_ref, k_ref, v_ref, o_ref):
    h = pl.program_id(0)
    ap = ap_ref[h]
    an = an_ref[h]
    q = q_ref[0] * jnp.bfloat16(SCALE)
    k = k_ref[0]
    s = lax.dot_general(q, k, (((1,), (1,)), ((), ())),
                        preferred_element_type=jnp.float32,
                        precision=_PREC)
    d = fk_ref[...] - fq_ref[...]
    s = s + d * jnp.where(d > 0.0, ap, an)
    m = jnp.max(s, axis=-1, keepdims=True)
    p = jnp.exp(s - m)
    p = p * (1.0 / jnp.sum(p, axis=-1, keepdims=True))
    o_ref[0] = jnp.dot(p.astype(jnp.bfloat16), v_ref[0],
                       preferred_element_type=jnp.float32, precision=_PREC)


def _attn_call(alpha_pos, alpha_neg, frac_col, frac_row, qkv3):
    return pl.pallas_call(
        _attn_body,
        grid=(H, T // BQ),
        in_specs=[
            pl.BlockSpec(memory_space=pltpu.SMEM),
            pl.BlockSpec(memory_space=pltpu.SMEM),
            pl.BlockSpec((BQ, 1), lambda h, qi: (qi, 0)),
            pl.BlockSpec((1, T), lambda h, qi: (0, 0)),
            pl.BlockSpec((1, BQ, HD), lambda h, qi: (h, qi, 0)),
            pl.BlockSpec((1, T, HD), lambda h, qi: (H + h, 0, 0)),
            pl.BlockSpec((1, T, HD), lambda h, qi: (2 * H + h, 0, 0)),
        ],
        out_specs=pl.BlockSpec((1, BQ, HD), lambda h, qi: (h, qi, 0)),
        out_shape=jax.ShapeDtypeStruct((H, T, HD), jnp.float32),
    )(alpha_pos, alpha_neg, frac_col, frac_row, qkv3, qkv3, qkv3)


def _post_body(ao_ref, src_ref, wo_ref, bo_ref, g_ref, be_ref, gw_ref, gb_ref,
               x_ref, lg_ref):
    r = (
        lax.dot_general(ao_ref[...], wo_ref[...], (((1,), (1,)), ((), ())),
                        preferred_element_type=jnp.float32, precision=_PREC)
        + bo_ref[...]
        + src_ref[...]
    )
    m = jnp.mean(r, axis=-1, keepdims=True)
    c = r - m
    v = jnp.mean(c * c, axis=-1, keepdims=True)
    x = c / jnp.sqrt(v + EPS) * g_ref[...] + be_ref[...]
    x_ref[...] = x
    lg_ref[...] = (
        lax.dot_general(x, gw_ref[...], (((1,), (1,)), ((), ())),
                        preferred_element_type=jnp.float32, precision=_PREC)
        + gb_ref[...]
    )


def _post_call(ao, src2, wot, bo, g1, beta1, gwpad, gbpad):
    return pl.pallas_call(
        _post_body,
        grid=(T // BR,),
        in_specs=[
            pl.BlockSpec((BR, D), lambda i: (i, 0)),
            pl.BlockSpec((BR, D), lambda i: (i, 0)),
            pl.BlockSpec((D, D), lambda i: (0, 0)),
            pl.BlockSpec((1, D), lambda i: (0, 0)),
            pl.BlockSpec((1, D), lambda i: (0, 0)),
            pl.BlockSpec((1, D), lambda i: (0, 0)),
            pl.BlockSpec((128, D), lambda i: (0, 0)),
            pl.BlockSpec((1, 128), lambda i: (0, 0)),
        ],
        out_specs=[
            pl.BlockSpec((BR, D), lambda i: (i, 0)),
            pl.BlockSpec((BR, 128), lambda i: (i, 0)),
        ],
        out_shape=[
            jax.ShapeDtypeStruct((T, D), jnp.float32),
            jax.ShapeDtypeStruct((T, 128), jnp.float32),
        ],
    )(ao, src2, wot, bo, g1, beta1, gwpad, gbpad)


def _route_body(lg_ref, dest_ref, s0_ref, s1_ref, te_ref, nt_ref):
    lg = lg_ref[...]
    lane = lax.broadcasted_iota(jnp.int32, (T, 128), 1)
    lane_f = lane.astype(jnp.float32)
    m = jnp.max(lg, axis=-1, keepdims=True)
    ex = jnp.exp(lg - m)
    sc = ex / jnp.sum(ex, axis=-1, keepdims=True)
    s0 = jnp.max(sc, axis=-1, keepdims=True)
    e0 = jnp.min(jnp.where(sc >= s0, lane, 1 << 30), axis=-1, keepdims=True)
    sc2 = jnp.where(lane == e0, -1.0, sc)
    s1 = jnp.max(sc2, axis=-1, keepdims=True)
    e1 = jnp.min(jnp.where(sc2 >= s1, lane, 1 << 30), axis=-1, keepdims=True)
    s0_ref[...] = s0
    s1_ref[...] = s1

    e_pair = jnp.concatenate([e0, e1], axis=0)
    lane4 = lax.broadcasted_iota(jnp.int32, (NPAIR, 128), 1)
    mh = (lane4 == e_pair).astype(jnp.float32)

    r0 = lax.broadcasted_iota(jnp.int32, (512, 512), 0)
    c0 = lax.broadcasted_iota(jnp.int32, (512, 512), 1)
    ltri = (r0 > c0).astype(jnp.float32)
    carry = jnp.zeros((1, 128), jnp.float32)
    pos_blocks = []
    m_blocks = []
    for i in range(8):
        mi = mh[i * 512:(i + 1) * 512]
        ci = jnp.dot(ltri, mi, preferred_element_type=jnp.float32,
                     precision=_XPREC) + carry
        pos_blocks.append(jnp.sum(ci * mi, axis=-1, keepdims=True))
        m_blocks.append(mi)
        carry = carry + jnp.sum(mi, axis=0, keepdims=True)
    pos = jnp.concatenate(pos_blocks, axis=0)
    counts = carry

    ntiles = (counts.astype(jnp.int32) + (TILE - 1)) >> 7
    ntf = ntiles.astype(jnp.float32)
    ru = lax.broadcasted_iota(jnp.int32, (128, 128), 0)
    cu = lax.broadcasted_iota(jnp.int32, (128, 128), 1)
    utri = (ru < cu).astype(jnp.float32)
    tstart = jnp.dot(ntf, utri, preferred_element_type=jnp.float32,
                     precision=_XPREC)
    pstart = tstart * float(TILE)

    pstart_pair = jnp.sum(mh * pstart, axis=-1, keepdims=True)
    dest_ref[...] = (pstart_pair + pos).astype(jnp.int32)

    ji = lax.broadcasted_iota(jnp.int32, (64, 128), 0).astype(jnp.float32)
    inseg = jnp.logical_and(ji >= tstart, ji < tstart + ntf).astype(jnp.float32)
    te_f = jnp.sum(inseg * lane_f[:64], axis=-1, keepdims=True)
    valid = jnp.sum(inseg, axis=-1, keepdims=True) > 0.0
    last_e = jnp.max(jnp.where(counts > 0.0, lane_f[:1], 0.0), axis=-1,
                     keepdims=True)
    te_ref[...] = jnp.where(valid, te_f, last_e).astype(jnp.int32)
    nt_ref[...] = jnp.sum(ntf, axis=-1, keepdims=True).astype(jnp.int32)


def _route_call(logits):
    return pl.pallas_call(
        _route_body,
        grid=(1,),
        in_specs=[pl.BlockSpec((T, 128), lambda i: (0, 0))],
        out_specs=[
            pl.BlockSpec((NPAIR, 1), lambda i: (0, 0)),
            pl.BlockSpec((T, 1), lambda i: (0, 0)),
            pl.BlockSpec((T, 1), lambda i: (0, 0)),
            pl.BlockSpec((64, 1), lambda i: (0, 0)),
            pl.BlockSpec((1, 1), lambda i: (0, 0)),
        ],
        out_shape=[
            jax.ShapeDtypeStruct((NPAIR, 1), jnp.int32),
            jax.ShapeDtypeStruct((T, 1), jnp.float32),
            jax.ShapeDtypeStruct((T, 1), jnp.float32),
            jax.ShapeDtypeStruct((64, 1), jnp.int32),
            jax.ShapeDtypeStruct((1, 1), jnp.int32),
        ],
    )(logits)


def _sc_mesh():
    return plsc.VectorSubcoreMesh(core_axis_name="c", subcore_axis_name="s")


_NW = 32
_PPW = NPAIR // _NW
_CH = 64


def _sc_scatter(x, dest):
    @functools.partial(
        pl.kernel,
        out_type=jax.ShapeDtypeStruct((PADDED, D), jnp.float32),
        mesh=_sc_mesh(),
        scratch_types=[
            pltpu.VMEM((_CH,), jnp.int32),
            pltpu.VMEM((_CH, D), jnp.float32),
            pltpu.SemaphoreType.DMA,
        ],
    )
    def body(x_hbm, dest_hbm, xs_hbm, idx_v, rows_v, sem):
        wid = lax.axis_index("s") * 2 + lax.axis_index("c")
        for chunk in range(_PPW // _CH):
            base = wid * _PPW + chunk * _CH
            tok = jnp.where(base >= T, base - T, base)
            pltpu.sync_copy(dest_hbm.at[pl.ds(base, _CH)], idx_v)
            pltpu.sync_copy(x_hbm.at[pl.ds(tok, _CH)], rows_v)
            pltpu.async_copy(rows_v, xs_hbm.at[idx_v], sem).wait()

    return body(x, dest)


def _sc_gather(hs, dest):
    @functools.partial(
        pl.kernel,
        out_type=jax.ShapeDtypeStruct((NPAIR, D), jnp.float32),
        mesh=_sc_mesh(),
        scratch_types=[
            pltpu.VMEM((_CH,), jnp.int32),
            pltpu.VMEM((_CH, D), jnp.float32),
            pltpu.SemaphoreType.DMA,
        ],
    )
    def body(hs_hbm, dest_hbm, hh_hbm, idx_v, rows_v, sem):
        wid = lax.axis_index("s") * 2 + lax.axis_index("c")
        for chunk in range(_PPW // _CH):
            base = wid * _PPW + chunk * _CH
            pltpu.sync_copy(dest_hbm.at[pl.ds(base, _CH)], idx_v)
            pltpu.async_copy(hs_hbm.at[idx_v], rows_v, sem).wait()
            pltpu.sync_copy(rows_v, hh_hbm.at[pl.ds(base, _CH)])

    return body(hs, dest)


def _ffn_body(te_ref, nt_ref, xs_ref, w1_ref, b1_ref, w2_ref, b2_ref, o_ref):
    j = pl.program_id(0)

    @pl.when(j < nt_ref[0])
    def _():
        xv = xs_ref[...]
        h = jnp.maximum(
            lax.dot_general(xv, w1_ref[0], (((1,), (1,)), ((), ())),
                            preferred_element_type=jnp.float32,
                            precision=_PREC) + b1_ref[0],
            0.0,
        )
        o_ref[...] = lax.dot_general(h, w2_ref[0], (((1,), (1,)), ((), ())),
                                     preferred_element_type=jnp.float32,
                                     precision=_PREC) + b2_ref[0]


def _ffn_call(te, nt, xs, w1, b1, w2, b2):
    def _jc(j, te_ref, nt_ref):
        return jnp.minimum(j, nt_ref[0] - 1)

    grid_spec = pltpu.PrefetchScalarGridSpec(
        num_scalar_prefetch=2,
        grid=(NTILES,),
        in_specs=[
            pl.BlockSpec((TILE, D), lambda j, te, nt: (_jc(j, te, nt), 0)),
            pl.BlockSpec((1, DFF, D), lambda j, te, nt: (te[_jc(j, te, nt)], 0, 0)),
            pl.BlockSpec((1, 1, DFF), lambda j, te, nt: (te[_jc(j, te, nt)], 0, 0)),
            pl.BlockSpec((1, D, DFF), lambda j, te, nt: (te[_jc(j, te, nt)], 0, 0)),
            pl.BlockSpec((1, 1, D), lambda j, te, nt: (te[_jc(j, te, nt)], 0, 0)),
        ],
        out_specs=pl.BlockSpec((TILE, D), lambda j, te, nt: (_jc(j, te, nt), 0)),
    )
    return pl.pallas_call(
        _ffn_body,
        grid_spec=grid_spec,
        out_shape=jax.ShapeDtypeStruct((PADDED, D), jnp.float32),
    )(te, nt, xs, w1, b1, w2, b2)


def _combine_body(x_ref, h0_ref, h1_ref, s0_ref, s1_ref, g_ref, be_ref, y_ref):
    z = x_ref[...] + s0_ref[...] * h0_ref[...] + s1_ref[...] * h1_ref[...]
    m = jnp.mean(z, axis=-1, keepdims=True)
    c = z - m
    v = jnp.mean(c * c, axis=-1, keepdims=True)
    y_ref[...] = c / jnp.sqrt(v + EPS) * g_ref[...] + be_ref[...]


def _combine_call(x, hh, s0, s1, g2, beta2):
    return pl.pallas_call(
        _combine_body,
        grid=(T // BR,),
        in_specs=[
            pl.BlockSpec((BR, D), lambda i: (i, 0)),
            pl.BlockSpec((BR, D), lambda i: (i, 0)),
            pl.BlockSpec((BR, D), lambda i: (i + T // BR, 0)),
            pl.BlockSpec((BR, 1), lambda i: (i, 0)),
            pl.BlockSpec((BR, 1), lambda i: (i, 0)),
            pl.BlockSpec((1, D), lambda i: (0, 0)),
            pl.BlockSpec((1, D), lambda i: (0, 0)),
        ],
        out_specs=pl.BlockSpec((BR, D), lambda i: (i, 0)),
        out_shape=jax.ShapeDtypeStruct((T, D), jnp.float32),
    )(x, hh, hh, s0, s1, g2, beta2)


def kernel(src, frac, Wq, bq, Wk, bk, Wv, bv, Wo, bo, alpha_pos, alpha_neg,
           gate_W, gate_b, W1, b1, W2, b2, g1, beta1, g2, beta2):
    src2 = src.reshape(T, D)
    frac_col = frac.reshape(T, 1)
    frac_row = frac.reshape(1, T)

    b3 = jnp.stack([bq, bk, bv]).reshape(3, 1, D)
    qkv = _qkv_call(src2, Wq, Wk, Wv, b3)
    qkv3 = qkv.reshape(T, 3 * H, HD).transpose(1, 0, 2)

    ao3 = _attn_call(alpha_pos, alpha_neg, frac_col, frac_row, qkv3)
    ao = ao3.transpose(1, 0, 2).reshape(T, D)

    gwpad = jnp.pad(gate_W, ((0, 128 - E), (0, 0)))
    gbpad = jnp.concatenate(
        [gate_b, jnp.full((128 - E,), -1e9, jnp.float32)]).reshape(1, 128)
    x, logits = _post_call(ao, src2, Wo, bo.reshape(1, D),
                           g1.reshape(1, D), beta1.reshape(1, D), gwpad, gbpad)

    dest2, s0, s1, te2, nt2 = _route_call(logits)
    dest = dest2.reshape(NPAIR)

    xs = _sc_scatter(x, dest)
    hs = _ffn_call(te2.reshape(64), nt2.reshape(1), xs,
                   W1, b1.reshape(E, 1, DFF), W2, b2.reshape(E, 1, D))
    hh = _sc_gather(hs, dest)

    y = _combine_call(x, hh, s0, s1, g2.reshape(1, D), beta2.reshape(1, D))
    return y.reshape(1, T, D)


# speedup vs baseline: 1.1205x; 1.1205x over previous
import functools

import jax
import jax.numpy as jnp
from jax import lax
from jax.experimental import pallas as pl
from jax.experimental.pallas import tpu as pltpu
from jax.experimental.pallas import tpu_sc as plsc

T = 2048
D = 1024
H = 16
HD = 64
DFF = 2048
E = 8
EPS = 1e-5
SCALE = HD ** -0.5

TILE = 256
NTILES = 24
PADDED = NTILES * TILE
NPAIR = 2 * T

BQ = 1024
BR = 512

_PREC = None
_XPREC = jax.lax.Precision.HIGHEST


def _qkv_body(x_ref, wq_ref, wk_ref, wv_ref, b_ref, o_ref):
    j = pl.program_id(1)
    for idx in range(3):
        w_ref = (wq_ref, wk_ref, wv_ref)[idx]

        @pl.when(j == idx)
        def _(w_ref=w_ref):
            o_ref[...] = (
                lax.dot_general(x_ref[...], w_ref[...],
                                (((1,), (1,)), ((), ())),
                                preferred_element_type=jnp.float32,
                                precision=_PREC)
                + b_ref[0]
            ).astype(jnp.bfloat16)


def _qkv_call(x, wq, wk, wv, b3):
    return pl.pallas_call(
        _qkv_body,
        grid=(T // BR, 3),
        in_specs=[
            pl.BlockSpec((BR, D), lambda i, j: (i, 0)),
            pl.BlockSpec((D, D), lambda i, j: (0, 0)),
            pl.BlockSpec((D, D), lambda i, j: (0, 0)),
            pl.BlockSpec((D, D), lambda i, j: (0, 0)),
            pl.BlockSpec((1, 1, D), lambda i, j: (j, 0, 0)),
        ],
        out_specs=pl.BlockSpec((BR, D), lambda i, j: (i, j)),
        out_shape=jax.ShapeDtypeStruct((T, 3 * D), jnp.bfloat16),
    )(x, wq, wk, wv, b3)


def _attn_body(ap_ref, an_ref, fq_ref, fk_ref, q_ref, k_ref, v_ref, o_ref):
    h = pl.program_id(0)
    ap = ap_ref[h]
    an = an_ref[h]
    q = q_ref[0] * jnp.bfloat16(SCALE)
    k = k_ref[0]
    s = lax.dot_general(q, k, (((1,), (1,)), ((), ())),
                        preferred_element_type=jnp.float32,
                        precision=_PREC)
    d = fk_ref[...] - fq_ref[...]
    s = s + d * jnp.where(d > 0.0, ap, an)
    m = jnp.max(s, axis=-1, keepdims=True)
    p = jnp.exp(s - m)
    p = p * (1.0 / jnp.sum(p, axis=-1, keepdims=True))
    o_ref[0] = jnp.dot(p.astype(jnp.bfloat16), v_ref[0],
                       preferred_element_type=jnp.float32, precision=_PREC)


def _attn_call(alpha_pos, alpha_neg, frac_col, frac_row, qkv3):
    return pl.pallas_call(
        _attn_body,
        grid=(H, T // BQ),
        in_specs=[
            pl.BlockSpec(memory_space=pltpu.SMEM),
            pl.BlockSpec(memory_space=pltpu.SMEM),
            pl.BlockSpec((BQ, 1), lambda h, qi: (qi, 0)),
            pl.BlockSpec((1, T), lambda h, qi: (0, 0)),
            pl.BlockSpec((1, BQ, HD), lambda h, qi: (h, qi, 0)),
            pl.BlockSpec((1, T, HD), lambda h, qi: (H + h, 0, 0)),
            pl.BlockSpec((1, T, HD), lambda h, qi: (2 * H + h, 0, 0)),
        ],
        out_specs=pl.BlockSpec((1, BQ, HD), lambda h, qi: (h, qi, 0)),
        out_shape=jax.ShapeDtypeStruct((H, T, HD), jnp.float32),
    )(alpha_pos, alpha_neg, frac_col, frac_row, qkv3, qkv3, qkv3)


def _post_body(ao_ref, src_ref, wo_ref, bo_ref, g_ref, be_ref, gw_ref, gb_ref,
               x_ref, dest_ref, s0_ref, s1_ref, te_ref, nt_ref, lgs_ref):
    i = pl.program_id(0)

    @pl.when(i < T // BR)
    def _():
        r = (
            lax.dot_general(ao_ref[...], wo_ref[...], (((1,), (1,)), ((), ())),
                            preferred_element_type=jnp.float32, precision=_PREC)
            + bo_ref[...]
            + src_ref[...]
        )
        m = jnp.mean(r, axis=-1, keepdims=True)
        c = r - m
        v = jnp.mean(c * c, axis=-1, keepdims=True)
        x = c / jnp.sqrt(v + EPS) * g_ref[...] + be_ref[...]
        x_ref[...] = x
        lgs_ref[pl.ds(i * BR, BR), :] = (
            lax.dot_general(x, gw_ref[...], (((1,), (1,)), ((), ())),
                            preferred_element_type=jnp.float32, precision=_PREC)
            + gb_ref[...]
        )

    @pl.when(i == T // BR)
    def _():
        lg = lgs_ref[...]
        lane = lax.broadcasted_iota(jnp.int32, (T, 128), 1)
        lane_f = lane.astype(jnp.float32)
        m = jnp.max(lg, axis=-1, keepdims=True)
        ex = jnp.exp(lg - m)
        sc = ex / jnp.sum(ex, axis=-1, keepdims=True)
        s0 = jnp.max(sc, axis=-1, keepdims=True)
        e0 = jnp.min(jnp.where(sc >= s0, lane, 1 << 30), axis=-1, keepdims=True)
        sc2 = jnp.where(lane == e0, -1.0, sc)
        s1 = jnp.max(sc2, axis=-1, keepdims=True)
        e1 = jnp.min(jnp.where(sc2 >= s1, lane, 1 << 30), axis=-1, keepdims=True)
        s0_ref[...] = s0
        s1_ref[...] = s1

        e_pair = jnp.concatenate([e0, e1], axis=0)
        lane4 = lax.broadcasted_iota(jnp.int32, (NPAIR, 128), 1)
        mh = (lane4 == e_pair).astype(jnp.float32)

        r0 = lax.broadcasted_iota(jnp.int32, (512, 512), 0)
        c0 = lax.broadcasted_iota(jnp.int32, (512, 512), 1)
        ltri = (r0 > c0).astype(jnp.float32)
        carry = jnp.zeros((1, 128), jnp.float32)
        pos_blocks = []
        for b in range(8):
            mi = mh[b * 512:(b + 1) * 512]
            ci = jnp.dot(ltri, mi, preferred_element_type=jnp.float32,
                         precision=_XPREC) + carry
            pos_blocks.append(jnp.sum(ci * mi, axis=-1, keepdims=True))
            carry = carry + jnp.sum(mi, axis=0, keepdims=True)
        pos = jnp.concatenate(pos_blocks, axis=0)
        counts = carry

        ntiles = (counts.astype(jnp.int32) + (TILE - 1)) >> 8
        ntf = ntiles.astype(jnp.float32)
        ru = lax.broadcasted_iota(jnp.int32, (128, 128), 0)
        cu = lax.broadcasted_iota(jnp.int32, (128, 128), 1)
        utri = (ru < cu).astype(jnp.float32)
        tstart = jnp.dot(ntf, utri, preferred_element_type=jnp.float32,
                         precision=_XPREC)
        pstart = tstart * float(TILE)

        pstart_pair = jnp.sum(mh * pstart, axis=-1, keepdims=True)
        dest_ref[...] = (pstart_pair + pos).astype(jnp.int32)

        ji = lax.broadcasted_iota(jnp.int32, (64, 128), 0).astype(jnp.float32)
        inseg = jnp.logical_and(ji >= tstart,
                                ji < tstart + ntf).astype(jnp.float32)
        te_f = jnp.sum(inseg * lane_f[:64], axis=-1, keepdims=True)
        valid = jnp.sum(inseg, axis=-1, keepdims=True) > 0.0
        last_e = jnp.max(jnp.where(counts > 0.0, lane_f[:1], 0.0), axis=-1,
                         keepdims=True)
        te_ref[...] = jnp.where(valid, te_f, last_e).astype(jnp.int32)
        nt_ref[...] = jnp.sum(ntf, axis=-1, keepdims=True).astype(jnp.int32)


def _post_call(ao, src2, wot, bo, g1, beta1, gwpad, gbpad):
    nb = T // BR

    def _rc(i):
        return (jnp.minimum(i, nb - 1), 0)

    return pl.pallas_call(
        _post_body,
        grid=(nb + 1,),
        in_specs=[
            pl.BlockSpec((BR, D), _rc),
            pl.BlockSpec((BR, D), _rc),
            pl.BlockSpec((D, D), lambda i: (0, 0)),
            pl.BlockSpec((1, D), lambda i: (0, 0)),
            pl.BlockSpec((1, D), lambda i: (0, 0)),
            pl.BlockSpec((1, D), lambda i: (0, 0)),
            pl.BlockSpec((128, D), lambda i: (0, 0)),
            pl.BlockSpec((1, 128), lambda i: (0, 0)),
        ],
        out_specs=[
            pl.BlockSpec((BR, D), _rc),
            pl.BlockSpec((NPAIR, 1), lambda i: (0, 0)),
            pl.BlockSpec((T, 1), lambda i: (0, 0)),
            pl.BlockSpec((T, 1), lambda i: (0, 0)),
            pl.BlockSpec((64, 1), lambda i: (0, 0)),
            pl.BlockSpec((1, 1), lambda i: (0, 0)),
        ],
        out_shape=[
            jax.ShapeDtypeStruct((T, D), jnp.float32),
            jax.ShapeDtypeStruct((NPAIR, 1), jnp.int32),
            jax.ShapeDtypeStruct((T, 1), jnp.float32),
            jax.ShapeDtypeStruct((T, 1), jnp.float32),
            jax.ShapeDtypeStruct((64, 1), jnp.int32),
            jax.ShapeDtypeStruct((1, 1), jnp.int32),
        ],
        scratch_shapes=[pltpu.VMEM((T, 128), jnp.float32)],
    )(ao, src2, wot, bo, g1, beta1, gwpad, gbpad)


def _sc_mesh():
    return plsc.VectorSubcoreMesh(core_axis_name="c", subcore_axis_name="s")


_NW = 32
_PPW = NPAIR // _NW
_CH = 64


def _sc_scatter(x, dest):
    @functools.partial(
        pl.kernel,
        out_type=jax.ShapeDtypeStruct((PADDED, D), jnp.float32),
        mesh=_sc_mesh(),
        scratch_types=[
            pltpu.VMEM((_CH,), jnp.int32),
            pltpu.VMEM((_CH, D), jnp.float32),
            pltpu.SemaphoreType.DMA,
        ],
    )
    def body(x_hbm, dest_hbm, xs_hbm, idx_v, rows_v, sem):
        wid = lax.axis_index("s") * 2 + lax.axis_index("c")
        for chunk in range(_PPW // _CH):
            base = wid * _PPW + chunk * _CH
            tok = jnp.where(base >= T, base - T, base)
            pltpu.sync_copy(dest_hbm.at[pl.ds(base, _CH)], idx_v)
            pltpu.sync_copy(x_hbm.at[pl.ds(tok, _CH)], rows_v)
            pltpu.async_copy(rows_v, xs_hbm.at[idx_v], sem).wait()

    return body(x, dest)


def _sc_gather(hs, dest):
    @functools.partial(
        pl.kernel,
        out_type=jax.ShapeDtypeStruct((NPAIR, D), jnp.float32),
        mesh=_sc_mesh(),
        scratch_types=[
            pltpu.VMEM((_CH,), jnp.int32),
            pltpu.VMEM((_CH, D), jnp.float32),
            pltpu.SemaphoreType.DMA,
        ],
    )
    def body(hs_hbm, dest_hbm, hh_hbm, idx_v, rows_v, sem):
        wid = lax.axis_index("s") * 2 + lax.axis_index("c")
        for chunk in range(_PPW // _CH):
            base = wid * _PPW + chunk * _CH
            pltpu.sync_copy(dest_hbm.at[pl.ds(base, _CH)], idx_v)
            pltpu.async_copy(hs_hbm.at[idx_v], rows_v, sem).wait()
            pltpu.sync_copy(rows_v, hh_hbm.at[pl.ds(base, _CH)])

    return body(hs, dest)


def _ffn_body(te_ref, nt_ref, xs_ref, w1_ref, b1_ref, w2_ref, b2_ref, o_ref):
    j = pl.program_id(0)

    @pl.when(j < nt_ref[0])
    def _():
        xv = xs_ref[...]
        h = jnp.maximum(
            lax.dot_general(xv, w1_ref[0], (((1,), (1,)), ((), ())),
                            preferred_element_type=jnp.float32,
                            precision=_PREC) + b1_ref[0],
            0.0,
        )
        o_ref[...] = lax.dot_general(h, w2_ref[0], (((1,), (1,)), ((), ())),
                                     preferred_element_type=jnp.float32,
                                     precision=_PREC) + b2_ref[0]


def _ffn_call(te, nt, xs, w1, b1, w2, b2):
    def _jc(j, te_ref, nt_ref):
        return jnp.minimum(j, nt_ref[0] - 1)

    grid_spec = pltpu.PrefetchScalarGridSpec(
        num_scalar_prefetch=2,
        grid=(NTILES,),
        in_specs=[
            pl.BlockSpec((TILE, D), lambda j, te, nt: (_jc(j, te, nt), 0)),
            pl.BlockSpec((1, DFF, D), lambda j, te, nt: (te[_jc(j, te, nt)], 0, 0)),
            pl.BlockSpec((1, 1, DFF), lambda j, te, nt: (te[_jc(j, te, nt)], 0, 0)),
            pl.BlockSpec((1, D, DFF), lambda j, te, nt: (te[_jc(j, te, nt)], 0, 0)),
            pl.BlockSpec((1, 1, D), lambda j, te, nt: (te[_jc(j, te, nt)], 0, 0)),
        ],
        out_specs=pl.BlockSpec((TILE, D), lambda j, te, nt: (_jc(j, te, nt), 0)),
    )
    return pl.pallas_call(
        _ffn_body,
        grid_spec=grid_spec,
        out_shape=jax.ShapeDtypeStruct((PADDED, D), jnp.float32),
    )(te, nt, xs, w1, b1, w2, b2)


def _combine_body(x_ref, h0_ref, h1_ref, s0_ref, s1_ref, g_ref, be_ref, y_ref):
    z = x_ref[...] + s0_ref[...] * h0_ref[...] + s1_ref[...] * h1_ref[...]
    m = jnp.mean(z, axis=-1, keepdims=True)
    c = z - m
    v = jnp.mean(c * c, axis=-1, keepdims=True)
    y_ref[...] = c / jnp.sqrt(v + EPS) * g_ref[...] + be_ref[...]


def _combine_call(x, hh, s0, s1, g2, beta2):
    return pl.pallas_call(
        _combine_body,
        grid=(T // BR,),
        in_specs=[
            pl.BlockSpec((BR, D), lambda i: (i, 0)),
            pl.BlockSpec((BR, D), lambda i: (i, 0)),
            pl.BlockSpec((BR, D), lambda i: (i + T // BR, 0)),
            pl.BlockSpec((BR, 1), lambda i: (i, 0)),
            pl.BlockSpec((BR, 1), lambda i: (i, 0)),
            pl.BlockSpec((1, D), lambda i: (0, 0)),
            pl.BlockSpec((1, D), lambda i: (0, 0)),
        ],
        out_specs=pl.BlockSpec((BR, D), lambda i: (i, 0)),
        out_shape=jax.ShapeDtypeStruct((T, D), jnp.float32),
    )(x, hh, hh, s0, s1, g2, beta2)


def kernel(src, frac, Wq, bq, Wk, bk, Wv, bv, Wo, bo, alpha_pos, alpha_neg,
           gate_W, gate_b, W1, b1, W2, b2, g1, beta1, g2, beta2):
    src2 = src.reshape(T, D)
    frac_col = frac.reshape(T, 1)
    frac_row = frac.reshape(1, T)

    b3 = jnp.stack([bq, bk, bv]).reshape(3, 1, D)
    qkv = _qkv_call(src2, Wq, Wk, Wv, b3)
    qkv3 = qkv.reshape(T, 3 * H, HD).transpose(1, 0, 2)

    ao3 = _attn_call(alpha_pos, alpha_neg, frac_col, frac_row, qkv3)
    ao = ao3.transpose(1, 0, 2).reshape(T, D)

    gwpad = jnp.pad(gate_W, ((0, 128 - E), (0, 0)))
    gbpad = jnp.concatenate(
        [gate_b, jnp.full((128 - E,), -1e9, jnp.float32)]).reshape(1, 128)
    x, dest2, s0, s1, te2, nt2 = _post_call(
        ao, src2, Wo, bo.reshape(1, D), g1.reshape(1, D), beta1.reshape(1, D),
        gwpad, gbpad)
    dest = dest2.reshape(NPAIR)

    xs = _sc_scatter(x, dest)
    hs = _ffn_call(te2.reshape(64), nt2.reshape(1), xs,
                   W1, b1.reshape(E, 1, DFF), W2, b2.reshape(E, 1, D))
    hh = _sc_gather(hs, dest)

    y = _combine_call(x, hh, s0, s1, g2.reshape(1, D), beta2.reshape(1, D))
    return y.reshape(1, T, D)


# speedup vs baseline: 1.1370x; 1.0147x over previous
import functools

import jax
import jax.numpy as jnp
from jax import lax
from jax.experimental import pallas as pl
from jax.experimental.pallas import tpu as pltpu
from jax.experimental.pallas import tpu_sc as plsc

T = 2048
D = 1024
H = 16
HD = 64
DFF = 2048
E = 8
EPS = 1e-5
SCALE = HD ** -0.5

TILE = 256
NTILES = 24
PADDED = NTILES * TILE
NPAIR = 2 * T

BQ = 2048
BR = 512

_PREC = None
_XPREC = jax.lax.Precision.HIGHEST


def _qkv_body(x_ref, wq_ref, wk_ref, wv_ref, b_ref, o_ref):
    j = pl.program_id(1)
    for idx in range(3):
        w_ref = (wq_ref, wk_ref, wv_ref)[idx]

        @pl.when(j == idx)
        def _(w_ref=w_ref):
            o_ref[...] = (
                lax.dot_general(x_ref[...], w_ref[...],
                                (((1,), (1,)), ((), ())),
                                preferred_element_type=jnp.float32,
                                precision=_PREC)
                + b_ref[0]
            ).astype(jnp.bfloat16)


def _qkv_call(x, wq, wk, wv, b3):
    return pl.pallas_call(
        _qkv_body,
        grid=(T // BR, 3),
        in_specs=[
            pl.BlockSpec((BR, D), lambda i, j: (i, 0)),
            pl.BlockSpec((D, D), lambda i, j: (0, 0)),
            pl.BlockSpec((D, D), lambda i, j: (0, 0)),
            pl.BlockSpec((D, D), lambda i, j: (0, 0)),
            pl.BlockSpec((1, 1, D), lambda i, j: (j, 0, 0)),
        ],
        out_specs=pl.BlockSpec((BR, D), lambda i, j: (i, j)),
        out_shape=jax.ShapeDtypeStruct((T, 3 * D), jnp.bfloat16),
    )(x, wq, wk, wv, b3)


def _attn_body(ap_ref, an_ref, fq_ref, fk_ref, q_ref, k_ref, v_ref, o_ref):
    h = pl.program_id(0)
    ap = ap_ref[h]
    an = an_ref[h]
    q = q_ref[0] * jnp.bfloat16(SCALE)
    k = k_ref[0]
    s = lax.dot_general(q, k, (((1,), (1,)), ((), ())),
                        preferred_element_type=jnp.float32,
                        precision=_PREC)
    d = fk_ref[...] - fq_ref[...]
    s = s + d * jnp.where(d > 0.0, ap, an)
    m = jnp.max(s, axis=-1, keepdims=True)
    p = jnp.exp(s - m)
    p = p * (1.0 / jnp.sum(p, axis=-1, keepdims=True))
    o_ref[0] = jnp.dot(p.astype(jnp.bfloat16), v_ref[0],
                       preferred_element_type=jnp.float32, precision=_PREC)


def _attn_call(alpha_pos, alpha_neg, frac_col, frac_row, qkv3):
    return pl.pallas_call(
        _attn_body,
        grid=(H, T // BQ),
        in_specs=[
            pl.BlockSpec(memory_space=pltpu.SMEM),
            pl.BlockSpec(memory_space=pltpu.SMEM),
            pl.BlockSpec((BQ, 1), lambda h, qi: (qi, 0)),
            pl.BlockSpec((1, T), lambda h, qi: (0, 0)),
            pl.BlockSpec((1, BQ, HD), lambda h, qi: (h, qi, 0)),
            pl.BlockSpec((1, T, HD), lambda h, qi: (H + h, 0, 0)),
            pl.BlockSpec((1, T, HD), lambda h, qi: (2 * H + h, 0, 0)),
        ],
        out_specs=pl.BlockSpec((1, BQ, HD), lambda h, qi: (h, qi, 0)),
        out_shape=jax.ShapeDtypeStruct((H, T, HD), jnp.float32),
    )(alpha_pos, alpha_neg, frac_col, frac_row, qkv3, qkv3, qkv3)


def _post_body(ao_ref, src_ref, wo_ref, bo_ref, g_ref, be_ref, gw_ref, gb_ref,
               x_ref, dest_ref, s0_ref, s1_ref, te_ref, nt_ref, lgs_ref):
    i = pl.program_id(0)

    @pl.when(i < T // BR)
    def _():
        r = (
            lax.dot_general(ao_ref[...], wo_ref[...], (((1,), (1,)), ((), ())),
                            preferred_element_type=jnp.float32, precision=_PREC)
            + bo_ref[...]
            + src_ref[...]
        )
        m = jnp.mean(r, axis=-1, keepdims=True)
        c = r - m
        v = jnp.mean(c * c, axis=-1, keepdims=True)
        x = c / jnp.sqrt(v + EPS) * g_ref[...] + be_ref[...]
        x_ref[...] = x
        lgs_ref[pl.ds(i * BR, BR), :] = (
            lax.dot_general(x, gw_ref[...], (((1,), (1,)), ((), ())),
                            preferred_element_type=jnp.float32, precision=_PREC)
            + gb_ref[...]
        )

    @pl.when(i == T // BR)
    def _():
        lg = lgs_ref[...]
        lane = lax.broadcasted_iota(jnp.int32, (T, 128), 1)
        lane_f = lane.astype(jnp.float32)
        m = jnp.max(lg, axis=-1, keepdims=True)
        ex = jnp.exp(lg - m)
        sc = ex / jnp.sum(ex, axis=-1, keepdims=True)
        s0 = jnp.max(sc, axis=-1, keepdims=True)
        e0 = jnp.min(jnp.where(sc >= s0, lane, 1 << 30), axis=-1, keepdims=True)
        sc2 = jnp.where(lane == e0, -1.0, sc)
        s1 = jnp.max(sc2, axis=-1, keepdims=True)
        e1 = jnp.min(jnp.where(sc2 >= s1, lane, 1 << 30), axis=-1, keepdims=True)
        s0_ref[...] = s0
        s1_ref[...] = s1

        e_pair = jnp.concatenate([e0, e1], axis=0)
        lane4 = lax.broadcasted_iota(jnp.int32, (NPAIR, 128), 1)
        mh = (lane4 == e_pair).astype(jnp.float32)

        r0 = lax.broadcasted_iota(jnp.int32, (512, 512), 0)
        c0 = lax.broadcasted_iota(jnp.int32, (512, 512), 1)
        ltri = (r0 > c0).astype(jnp.float32)
        carry = jnp.zeros((1, 128), jnp.float32)
        pos_blocks = []
        for b in range(8):
            mi = mh[b * 512:(b + 1) * 512]
            ci = jnp.dot(ltri, mi, preferred_element_type=jnp.float32,
                         precision=_XPREC) + carry
            pos_blocks.append(jnp.sum(ci * mi, axis=-1, keepdims=True))
            carry = carry + jnp.sum(mi, axis=0, keepdims=True)
        pos = jnp.concatenate(pos_blocks, axis=0)
        counts = carry

        ntiles = (counts.astype(jnp.int32) + (TILE - 1)) >> 8
        ntf = ntiles.astype(jnp.float32)
        ru = lax.broadcasted_iota(jnp.int32, (128, 128), 0)
        cu = lax.broadcasted_iota(jnp.int32, (128, 128), 1)
        utri = (ru < cu).astype(jnp.float32)
        tstart = jnp.dot(ntf, utri, preferred_element_type=jnp.float32,
                         precision=_XPREC)
        pstart = tstart * float(TILE)

        pstart_pair = jnp.sum(mh * pstart, axis=-1, keepdims=True)
        dest_ref[...] = (pstart_pair + pos).astype(jnp.int32)

        ji = lax.broadcasted_iota(jnp.int32, (64, 128), 0).astype(jnp.float32)
        inseg = jnp.logical_and(ji >= tstart,
                                ji < tstart + ntf).astype(jnp.float32)
        te_f = jnp.sum(inseg * lane_f[:64], axis=-1, keepdims=True)
        valid = jnp.sum(inseg, axis=-1, keepdims=True) > 0.0
        last_e = jnp.max(jnp.where(counts > 0.0, lane_f[:1], 0.0), axis=-1,
                         keepdims=True)
        te_ref[...] = jnp.where(valid, te_f, last_e).astype(jnp.int32)
        nt_ref[...] = jnp.sum(ntf, axis=-1, keepdims=True).astype(jnp.int32)


def _post_call(ao, src2, wot, bo, g1, beta1, gwpad, gbpad):
    nb = T // BR

    def _rc(i):
        return (jnp.minimum(i, nb - 1), 0)

    return pl.pallas_call(
        _post_body,
        grid=(nb + 1,),
        in_specs=[
            pl.BlockSpec((BR, D), _rc),
            pl.BlockSpec((BR, D), _rc),
            pl.BlockSpec((D, D), lambda i: (0, 0)),
            pl.BlockSpec((1, D), lambda i: (0, 0)),
            pl.BlockSpec((1, D), lambda i: (0, 0)),
            pl.BlockSpec((1, D), lambda i: (0, 0)),
            pl.BlockSpec((128, D), lambda i: (0, 0)),
            pl.BlockSpec((1, 128), lambda i: (0, 0)),
        ],
        out_specs=[
            pl.BlockSpec((BR, D), _rc),
            pl.BlockSpec((NPAIR, 1), lambda i: (0, 0)),
            pl.BlockSpec((T, 1), lambda i: (0, 0)),
            pl.BlockSpec((T, 1), lambda i: (0, 0)),
            pl.BlockSpec((64, 1), lambda i: (0, 0)),
            pl.BlockSpec((1, 1), lambda i: (0, 0)),
        ],
        out_shape=[
            jax.ShapeDtypeStruct((T, D), jnp.float32),
            jax.ShapeDtypeStruct((NPAIR, 1), jnp.int32),
            jax.ShapeDtypeStruct((T, 1), jnp.float32),
            jax.ShapeDtypeStruct((T, 1), jnp.float32),
            jax.ShapeDtypeStruct((64, 1), jnp.int32),
            jax.ShapeDtypeStruct((1, 1), jnp.int32),
        ],
        scratch_shapes=[pltpu.VMEM((T, 128), jnp.float32)],
    )(ao, src2, wot, bo, g1, beta1, gwpad, gbpad)


def _sc_mesh():
    return plsc.VectorSubcoreMesh(core_axis_name="c", subcore_axis_name="s")


_NW = 32
_PPW = NPAIR // _NW
_CH = 64


def _sc_scatter(x, dest):
    @functools.partial(
        pl.kernel,
        out_type=jax.ShapeDtypeStruct((PADDED, D), jnp.float32),
        mesh=_sc_mesh(),
        scratch_types=[
            pltpu.VMEM((_CH,), jnp.int32),
            pltpu.VMEM((_CH, D), jnp.float32),
            pltpu.SemaphoreType.DMA,
        ],
    )
    def body(x_hbm, dest_hbm, xs_hbm, idx_v, rows_v, sem):
        wid = lax.axis_index("s") * 2 + lax.axis_index("c")
        for chunk in range(_PPW // _CH):
            base = wid * _PPW + chunk * _CH
            tok = jnp.where(base >= T, base - T, base)
            pltpu.sync_copy(dest_hbm.at[pl.ds(base, _CH)], idx_v)
            pltpu.sync_copy(x_hbm.at[pl.ds(tok, _CH)], rows_v)
            pltpu.async_copy(rows_v, xs_hbm.at[idx_v], sem).wait()

    return body(x, dest)


def _sc_gather(hs, dest):
    @functools.partial(
        pl.kernel,
        out_type=jax.ShapeDtypeStruct((NPAIR, D), jnp.float32),
        mesh=_sc_mesh(),
        scratch_types=[
            pltpu.VMEM((_CH,), jnp.int32),
            pltpu.VMEM((_CH, D), jnp.float32),
            pltpu.SemaphoreType.DMA,
        ],
    )
    def body(hs_hbm, dest_hbm, hh_hbm, idx_v, rows_v, sem):
        wid = lax.axis_index("s") * 2 + lax.axis_index("c")
        for chunk in range(_PPW // _CH):
            base = wid * _PPW + chunk * _CH
            pltpu.sync_copy(dest_hbm.at[pl.ds(base, _CH)], idx_v)
            pltpu.async_copy(hs_hbm.at[idx_v], rows_v, sem).wait()
            pltpu.sync_copy(rows_v, hh_hbm.at[pl.ds(base, _CH)])

    return body(hs, dest)


def _ffn_body(te_ref, nt_ref, xs_ref, w1_ref, b1_ref, w2_ref, b2_ref, o_ref):
    j = pl.program_id(0)

    @pl.when(j < nt_ref[0])
    def _():
        xv = xs_ref[...]
        h = jnp.maximum(
            lax.dot_general(xv, w1_ref[0], (((1,), (1,)), ((), ())),
                            preferred_element_type=jnp.float32,
                            precision=_PREC) + b1_ref[0],
            0.0,
        )
        o_ref[...] = lax.dot_general(h, w2_ref[0], (((1,), (1,)), ((), ())),
                                     preferred_element_type=jnp.float32,
                                     precision=_PREC) + b2_ref[0]


def _ffn_call(te, nt, xs, w1, b1, w2, b2):
    def _jc(j, te_ref, nt_ref):
        return jnp.minimum(j, nt_ref[0] - 1)

    grid_spec = pltpu.PrefetchScalarGridSpec(
        num_scalar_prefetch=2,
        grid=(NTILES,),
        in_specs=[
            pl.BlockSpec((TILE, D), lambda j, te, nt: (_jc(j, te, nt), 0)),
            pl.BlockSpec((1, DFF, D), lambda j, te, nt: (te[_jc(j, te, nt)], 0, 0)),
            pl.BlockSpec((1, 1, DFF), lambda j, te, nt: (te[_jc(j, te, nt)], 0, 0)),
            pl.BlockSpec((1, D, DFF), lambda j, te, nt: (te[_jc(j, te, nt)], 0, 0)),
            pl.BlockSpec((1, 1, D), lambda j, te, nt: (te[_jc(j, te, nt)], 0, 0)),
        ],
        out_specs=pl.BlockSpec((TILE, D), lambda j, te, nt: (_jc(j, te, nt), 0)),
    )
    return pl.pallas_call(
        _ffn_body,
        grid_spec=grid_spec,
        out_shape=jax.ShapeDtypeStruct((PADDED, D), jnp.float32),
    )(te, nt, xs, w1, b1, w2, b2)


def _combine_body(x_ref, h0_ref, h1_ref, s0_ref, s1_ref, g_ref, be_ref, y_ref):
    z = x_ref[...] + s0_ref[...] * h0_ref[...] + s1_ref[...] * h1_ref[...]
    m = jnp.mean(z, axis=-1, keepdims=True)
    c = z - m
    v = jnp.mean(c * c, axis=-1, keepdims=True)
    y_ref[...] = c / jnp.sqrt(v + EPS) * g_ref[...] + be_ref[...]


def _combine_call(x, hh, s0, s1, g2, beta2):
    return pl.pallas_call(
        _combine_body,
        grid=(T // BR,),
        in_specs=[
            pl.BlockSpec((BR, D), lambda i: (i, 0)),
            pl.BlockSpec((BR, D), lambda i: (i, 0)),
            pl.BlockSpec((BR, D), lambda i: (i + T // BR, 0)),
            pl.BlockSpec((BR, 1), lambda i: (i, 0)),
            pl.BlockSpec((BR, 1), lambda i: (i, 0)),
            pl.BlockSpec((1, D), lambda i: (0, 0)),
            pl.BlockSpec((1, D), lambda i: (0, 0)),
        ],
        out_specs=pl.BlockSpec((BR, D), lambda i: (i, 0)),
        out_shape=jax.ShapeDtypeStruct((T, D), jnp.float32),
    )(x, hh, hh, s0, s1, g2, beta2)


def kernel(src, frac, Wq, bq, Wk, bk, Wv, bv, Wo, bo, alpha_pos, alpha_neg,
           gate_W, gate_b, W1, b1, W2, b2, g1, beta1, g2, beta2):
    src2 = src.reshape(T, D)
    frac_col = frac.reshape(T, 1)
    frac_row = frac.reshape(1, T)

    b3 = jnp.stack([bq, bk, bv]).reshape(3, 1, D)
    qkv = _qkv_call(src2, Wq, Wk, Wv, b3)
    qkv3 = qkv.reshape(T, 3 * H, HD).transpose(1, 0, 2)

    ao3 = _attn_call(alpha_pos, alpha_neg, frac_col, frac_row, qkv3)
    ao = ao3.transpose(1, 0, 2).reshape(T, D)

    gwpad = jnp.pad(gate_W, ((0, 128 - E), (0, 0)))
    gbpad = jnp.concatenate(
        [gate_b, jnp.full((128 - E,), -1e9, jnp.float32)]).reshape(1, 128)
    x, dest2, s0, s1, te2, nt2 = _post_call(
        ao, src2, Wo, bo.reshape(1, D), g1.reshape(1, D), beta1.reshape(1, D),
        gwpad, gbpad)
    dest = dest2.reshape(NPAIR)

    xs = _sc_scatter(x, dest)
    hs = _ffn_call(te2.reshape(64), nt2.reshape(1), xs,
                   W1, b1.reshape(E, 1, DFF), W2, b2.reshape(E, 1, D))
    hh = _sc_gather(hs, dest)

    y = _combine_call(x, hh, s0, s1, g2.reshape(1, D), beta2.reshape(1, D))
    return y.reshape(1, T, D)
